# Initial kernel scaffold; baseline (speedup 1.0000x reference)
import jax
import jax.numpy as jnp
from jax.experimental import pallas as pl


def kernel(indices, weight):
    raise NotImplementedError("write your pallas kernel here")


# speedup vs baseline: 1.4593x; 1.4593x over previous
import functools

import jax
import jax.numpy as jnp
from jax import lax
from jax.experimental import pallas as pl
from jax.experimental.pallas import tpu as pltpu
from jax.experimental.pallas import tpu_sc as plsc

_NC = 2
_NS = 16
_NW = _NC * _NS

_CHUNK = 1024


@functools.partial(jax.jit, static_argnums=(2, 3))
def _sc_gather(idx_flat, weight, b_per_w, chunk):
    B = idx_flat.shape[0]
    D = weight.shape[1]
    n_steps = b_per_w // chunk

    mesh = plsc.VectorSubcoreMesh(
        core_axis_name="c", subcore_axis_name="s", num_cores=_NC,
        num_subcores=_NS)

    @functools.partial(
        pl.kernel,
        out_type=jax.ShapeDtypeStruct((B, D), jnp.float32),
        mesh=mesh,
        scratch_types=[
            pltpu.VMEM((chunk,), jnp.int32),
            pltpu.VMEM((chunk, D), jnp.float32),
            pltpu.SemaphoreType.DMA,
        ],
        compiler_params=pltpu.CompilerParams(use_tc_tiling_on_sc=False),
    )
    def k(idx_hbm, table_hbm, out_hbm, idx_v, rows_v, sem):
        wid = lax.axis_index("s") * _NC + lax.axis_index("c")
        base = wid * b_per_w

        def step(i, carry):
            off = base + i * chunk
            pltpu.sync_copy(idx_hbm.at[pl.ds(off, chunk)], idx_v)
            pltpu.async_copy(table_hbm.at[idx_v], rows_v, sem).wait()
            pltpu.sync_copy(rows_v, out_hbm.at[pl.ds(off, chunk)])
            return carry

        lax.fori_loop(0, n_steps, step, 0)

    return k(idx_flat, weight)


def kernel(indices, weight):
    B = indices.size
    D = weight.shape[1]
    idx_flat = indices.reshape(B).astype(jnp.int32)
    out = _sc_gather(idx_flat, weight, B // _NW, _CHUNK)
    return out.reshape(*indices.shape, D)


# speedup vs baseline: 1.4912x; 1.0219x over previous
import functools

import jax
import jax.numpy as jnp
from jax import lax
from jax.experimental import pallas as pl
from jax.experimental.pallas import tpu as pltpu
from jax.experimental.pallas import tpu_sc as plsc

_NC = 2
_NS = 16
_NW = _NC * _NS

_CHUNK = 1600


@functools.partial(jax.jit, static_argnums=(2, 3))
def _sc_gather(idx_flat, weight, b_per_w, chunk):
    B = idx_flat.shape[0]
    D = weight.shape[1]
    n_steps = b_per_w // chunk
    assert n_steps % 2 == 0 and n_steps >= 4

    mesh = plsc.VectorSubcoreMesh(
        core_axis_name="c", subcore_axis_name="s", num_cores=_NC,
        num_subcores=_NS)

    @functools.partial(
        pl.kernel,
        out_type=jax.ShapeDtypeStruct((B, D), jnp.float32),
        mesh=mesh,
        scratch_types=[
            pltpu.VMEM((chunk,), jnp.int32),
            pltpu.VMEM((chunk,), jnp.int32),
            pltpu.VMEM((chunk, D), jnp.float32),
            pltpu.VMEM((chunk, D), jnp.float32),
            pltpu.SemaphoreType.DMA,
            pltpu.SemaphoreType.DMA,
            pltpu.SemaphoreType.DMA,
            pltpu.SemaphoreType.DMA,
            pltpu.SemaphoreType.DMA,
            pltpu.SemaphoreType.DMA,
        ],
        compiler_params=pltpu.CompilerParams(use_tc_tiling_on_sc=False),
    )
    def k(idx_hbm, table_hbm, out_hbm, idx0, idx1, rows0, rows1,
          isem0, isem1, gsem0, gsem1, osem0, osem1):
        wid = lax.axis_index("s") * _NC + lax.axis_index("c")
        base = wid * b_per_w
        idx_b = (idx0, idx1)
        rows_b = (rows0, rows1)
        isem = (isem0, isem1)
        gsem = (gsem0, gsem1)
        osem = (osem0, osem1)

        def idx_start(g, b):
            off = jnp.minimum(base + g * chunk, B - chunk)
            return pltpu.async_copy(
                idx_hbm.at[pl.ds(off, chunk)], idx_b[b], isem[b])

        def idx_wait(b):
            pltpu.make_async_copy(
                idx_hbm.at[pl.ds(base, chunk)], idx_b[b], isem[b]).wait()

        def gather_start(b):
            return pltpu.async_copy(
                table_hbm.at[idx_b[b]], rows_b[b], gsem[b])

        def gather_wait(b):
            pltpu.make_async_copy(
                table_hbm.at[idx_b[b]], rows_b[b], gsem[b]).wait()

        def out_start(g, b):
            off = base + g * chunk
            return pltpu.async_copy(
                rows_b[b], out_hbm.at[pl.ds(off, chunk)], osem[b])

        def out_wait(b):
            pltpu.make_async_copy(
                rows_b[b], out_hbm.at[pl.ds(base, chunk)], osem[b]).wait()

        idx_start(0, 0).wait()
        gather_start(0)
        idx_start(1, 1)
        gather_wait(0)
        out_start(0, 0)
        idx_start(2, 0)
        idx_wait(1)
        gather_start(1)

        def superstep(t, carry):
            g = 2 * t
            gather_wait(1)
            out_start(g - 1, 1)
            idx_start(g + 1, 1)
            out_wait(0)
            idx_wait(0)
            gather_start(0)
            gather_wait(0)
            out_start(g, 0)
            idx_start(g + 2, 0)
            out_wait(1)
            idx_wait(1)
            gather_start(1)
            return carry

        lax.fori_loop(1, n_steps // 2, superstep, 0)

        gather_wait(1)
        out_start(n_steps - 1, 1)
        out_wait(0)
        idx_wait(0)
        out_wait(1)

    return k(idx_flat, weight)


def kernel(indices, weight):
    B = indices.size
    D = weight.shape[1]
    idx_flat = indices.reshape(B).astype(jnp.int32)
    out = _sc_gather(idx_flat, weight, B // _NW, _CHUNK)
    return out.reshape(*indices.shape, D)
